# Initial kernel scaffold; baseline (speedup 1.0000x reference)
import jax
import jax.numpy as jnp
from jax.experimental import pallas as pl


def kernel(position_ids, table):
    raise NotImplementedError("write your pallas kernel here")


# speedup vs baseline: 1.9806x; 1.9806x over previous
import functools

import jax
import jax.numpy as jnp
from jax import lax
from jax.experimental import pallas as pl
from jax.experimental.pallas import tpu as pltpu
from jax.experimental.pallas import tpu_sc as plsc

_NC, _NS = 2, 16
_NW = _NC * _NS


@functools.partial(jax.jit, static_argnums=(2, 3, 4))
def _sc_gather(ids, table, B, V, D):
    b_per_w = B // _NW
    chunk = 32
    n_chunks = b_per_w // chunk
    mesh = plsc.VectorSubcoreMesh(core_axis_name="c", subcore_axis_name="s")

    @functools.partial(
        pl.kernel,
        mesh=mesh,
        out_type=jax.ShapeDtypeStruct((B, D), jnp.float32),
        scratch_types=[
            pltpu.VMEM((b_per_w,), jnp.int32),
            pltpu.VMEM((chunk, D), jnp.float32),
            pltpu.SemaphoreType.DMA,
        ],
    )
    def k(idx_hbm, table_hbm, out_hbm, idx_v, rows_v, gsem):
        wid = lax.axis_index("s") * _NC + lax.axis_index("c")
        base = wid * b_per_w
        pltpu.sync_copy(idx_hbm.at[pl.ds(base, b_per_w)], idx_v)

        def body(c, _):
            idx_slice = idx_v.at[pl.ds(c * chunk, chunk)]
            pltpu.async_copy(table_hbm.at[idx_slice], rows_v, gsem).wait()
            pltpu.sync_copy(rows_v, out_hbm.at[pl.ds(base + c * chunk, chunk)])
            return 0

        lax.fori_loop(0, n_chunks, body, 0)

    return k(ids, table)


def kernel(position_ids, table):
    bsz, seq = position_ids.shape
    V, D = table.shape
    ids = position_ids.reshape(-1)
    out = _sc_gather(ids, table, bsz * seq, V, D)
    return out.reshape(bsz, seq, D)


# speedup vs baseline: 2.3800x; 1.2017x over previous
import functools

import jax
import jax.numpy as jnp
from jax import lax
from jax.experimental import pallas as pl
from jax.experimental.pallas import tpu as pltpu
from jax.experimental.pallas import tpu_sc as plsc

_NC, _NS = 2, 16
_NW = _NC * _NS


@functools.partial(jax.jit, static_argnums=(2, 3, 4))
def _sc_gather(ids, table, B, V, D):
    b_per_w = B // _NW
    chunk = 32
    n_chunks = b_per_w // chunk
    mesh = plsc.VectorSubcoreMesh(core_axis_name="c", subcore_axis_name="s")

    assert n_chunks % 2 == 0

    @functools.partial(
        pl.kernel,
        mesh=mesh,
        out_type=jax.ShapeDtypeStruct((B, D), jnp.float32),
        scratch_types=[
            pltpu.VMEM((b_per_w,), jnp.int32),
            pltpu.VMEM((chunk, D), jnp.float32),
            pltpu.VMEM((chunk, D), jnp.float32),
            pltpu.SemaphoreType.DMA,
            pltpu.SemaphoreType.DMA,
        ],
    )
    def k(idx_hbm, table_hbm, out_hbm, idx_v, rows0, rows1, g0, g1):
        wid = lax.axis_index("s") * _NC + lax.axis_index("c")
        base = wid * b_per_w
        pltpu.sync_copy(idx_hbm.at[pl.ds(base, b_per_w)], idx_v)

        def gather(c, buf, sem):
            idx_slice = idx_v.at[pl.ds(c * chunk, chunk)]
            return pltpu.make_async_copy(table_hbm.at[idx_slice], buf, sem)

        def writeback(c, buf):
            pltpu.sync_copy(buf, out_hbm.at[pl.ds(base + c * chunk, chunk)])

        gather(0, rows0, g0).start()

        def body(i, _):
            c0 = 2 * i
            gather(c0 + 1, rows1, g1).start()
            gather(c0, rows0, g0).wait()
            writeback(c0, rows0)

            @pl.when(c0 + 2 < n_chunks)
            def _():
                gather(c0 + 2, rows0, g0).start()

            gather(c0 + 1, rows1, g1).wait()
            writeback(c0 + 1, rows1)
            return 0

        lax.fori_loop(0, n_chunks // 2, body, 0)

    return k(ids, table)


def kernel(position_ids, table):
    bsz, seq = position_ids.shape
    V, D = table.shape
    ids = position_ids.reshape(-1)
    out = _sc_gather(ids, table, bsz * seq, V, D)
    return out.reshape(bsz, seq, D)
